# baseline (speedup 1.0000x reference)
import functools

import jax
import jax.numpy as jnp
from jax import lax
from jax.experimental import pallas as pl
from jax.experimental.pallas import tpu as pltpu
from jax.experimental.pallas import tpu_sc as plsc

NC, NS = 2, 16
NW = NC * NS
LANES = 16


@functools.lru_cache(maxsize=None)
def _make_sc_gather(b_total: int, hist: int, d_model: int):
    jt = b_total // 128
    j_per_w = jt // NW
    n_units = j_per_w * hist
    n_pairs = n_units // 2
    di = d_model // 8
    assert jt == NW * j_per_w and n_units == 2 * n_pairs
    mesh = plsc.VectorSubcoreMesh(core_axis_name="c", subcore_axis_name="s")

    @functools.partial(
        pl.kernel,
        out_type=jax.ShapeDtypeStruct((hist * di * jt * 8 * 128,),
                                      jnp.float32),
        mesh=mesh,
        scratch_types=[
            pltpu.VMEM((j_per_w * 128 * 128,), jnp.int32),
            pltpu.VMEM((2, 128), jnp.int32),
            pltpu.VMEM((2, 128, d_model), jnp.float32),
            pltpu.VMEM((2 * d_model * 128,), jnp.float32),
            pltpu.SemaphoreType.DMA,
            pltpu.SemaphoreType.DMA,
            pltpu.SemaphoreType.DMA,
            pltpu.SemaphoreType.DMA,
        ],
        compiler_params=pltpu.CompilerParams(use_tc_tiling_on_sc=False,
                                             needs_layout_passes=False,
                                             disable_bounds_checks=True),
    )
    def gather_kernel(x_hbm, table_hbm, out_hbm, idxj, idxcol, g, tp,
                      gs0, gs1, os0, os1):
        wid = lax.axis_index("s") * NC + lax.axis_index("c")
        jbase = wid * j_per_w
        pltpu.sync_copy(x_hbm.at[pl.ds(jbase * 128 * 128, j_per_w * 128 * 128)],
                        idxj)
        iota = lax.iota(jnp.int32, LANES)
        i128 = iota * 128

        def build_idxcol(u, s):
            base = (u // hist) * (128 * 128) + (u % hist)
            for cc in range(8):
                v = plsc.load_gather(idxj, [i128 + (base + cc * 2048)])
                idxcol[s, pl.ds(cc * LANES, LANES)] = v

        def fire_gather(s, gsem):
            pltpu.async_copy(table_hbm.at[idxcol.at[s]], g.at[s], gsem)

        def wait_gather(s, gsem):
            pltpu.make_async_copy(
                table_hbm.at[idxcol.at[s]], g.at[s], gsem).wait()

        def transpose(s):
            idxs = [(iota + dd * LANES) * 128 + s * (d_model * 128)
                    for dd in range(d_model // LANES)]
            for c in range(128):
                for dd in range(d_model // LANES):
                    v = g[s, c, pl.ds(dd * LANES, LANES)]
                    plsc.store_scatter(tp, [idxs[dd]], v)
                    idxs[dd] = idxs[dd] + 1

        def fire_out(u, s, osem):
            h = u % hist
            j = jbase + u // hist
            for i in range(di):
                pltpu.async_copy(
                    tp.at[pl.ds(s * d_model * 128 + i * 1024, 1024)],
                    out_hbm.at[pl.ds(((h * di + i) * jt + j) * 1024, 1024)],
                    osem)

        def wait_out(s, osem):
            pltpu.make_async_copy(
                out_hbm.at[pl.ds(0, 1024)], tp.at[pl.ds(0, 1024)],
                osem).wait()

        build_idxcol(0, 0)
        fire_gather(0, gs0)

        bufs = ((0, gs0, os0), (1, gs1, os1))

        def pair(k, carry):
            u0 = 2 * k
            for s, gsem, osem in bufs:
                u = u0 + s
                other = 1 - s
                ogsem = bufs[other][1]

                @pl.when(u + 1 < n_units)
                def _():
                    build_idxcol(u + 1, other)
                    fire_gather(other, ogsem)

                wait_gather(s, gsem)

                @pl.when(u >= 2)
                def _():
                    for _i in range(di):
                        wait_out(s, osem)

                transpose(s)
                fire_out(u, s, osem)
            return carry

        lax.fori_loop(0, n_pairs, pair, 0)
        for s, _g, osem in bufs:
            for _i in range(di):
                wait_out(s, osem)

    return gather_kernel


def kernel(x, table):
    b, h = x.shape
    d = table.shape[1]
    xp = jnp.pad(x.astype(jnp.int32), ((0, 0), (0, 128 - h))).reshape(-1)
    out1 = _make_sc_gather(b, h, d)(xp, table)
    out5 = out1.reshape(h, d // 8, b // 128, 8, 128)
    return out5.transpose(2, 4, 0, 1, 3).reshape(b, h, d)


# speedup vs baseline: 1.0045x; 1.0045x over previous
import functools

import jax
import jax.numpy as jnp
from jax import lax
from jax.experimental import pallas as pl
from jax.experimental.pallas import tpu as pltpu
from jax.experimental.pallas import tpu_sc as plsc

NC, NS = 2, 16
NW = NC * NS
LANES = 16


@functools.lru_cache(maxsize=None)
def _make_sc_gather(b_total: int, hist: int, d_model: int):
    jt = b_total // 128
    j_per_w = jt // NW
    n_units = j_per_w * hist
    n_quads = n_units // 4
    di = d_model // 8
    assert jt == NW * j_per_w and n_units == 4 * n_quads
    mesh = plsc.VectorSubcoreMesh(core_axis_name="c", subcore_axis_name="s")

    @functools.partial(
        pl.kernel,
        out_type=jax.ShapeDtypeStruct((hist * di * jt * 8 * 128,),
                                      jnp.float32),
        mesh=mesh,
        scratch_types=[
            pltpu.VMEM((j_per_w * 128 * 128,), jnp.int32),
            pltpu.VMEM((4, 128), jnp.int32),
            pltpu.VMEM((4, 128, d_model), jnp.float32),
            pltpu.VMEM((2 * d_model * 128,), jnp.float32),
            pltpu.SemaphoreType.DMA,
            pltpu.SemaphoreType.DMA,
            pltpu.SemaphoreType.DMA,
            pltpu.SemaphoreType.DMA,
            pltpu.SemaphoreType.DMA,
            pltpu.SemaphoreType.DMA,
        ],
        compiler_params=pltpu.CompilerParams(use_tc_tiling_on_sc=False,
                                             needs_layout_passes=False,
                                             disable_bounds_checks=True),
    )
    def gather_kernel(x_hbm, table_hbm, out_hbm, idxj, idxcol, g, tp,
                      gs0, gs1, gs2, gs3, os0, os1):
        wid = lax.axis_index("s") * NC + lax.axis_index("c")
        jbase = wid * j_per_w
        pltpu.sync_copy(x_hbm.at[pl.ds(jbase * 128 * 128, j_per_w * 128 * 128)],
                        idxj)
        iota = lax.iota(jnp.int32, LANES)
        i128 = iota * 128

        def build_idxcol(u, s):
            base = (u // hist) * (128 * 128) + (u % hist)
            for cc in range(8):
                v = plsc.load_gather(idxj, [i128 + (base + cc * 2048)])
                idxcol[s, pl.ds(cc * LANES, LANES)] = v

        def fire_gather(s, gsem):
            pltpu.async_copy(table_hbm.at[idxcol.at[s]], g.at[s], gsem)

        def wait_gather(s, gsem):
            pltpu.make_async_copy(
                table_hbm.at[idxcol.at[s]], g.at[s], gsem).wait()

        base_idx = [(iota + dd * LANES) * 128
                    for dd in range(d_model // LANES)]

        def transpose(s, s2):
            off = s2 * (d_model * 128)

            def body(k, carry):
                c0 = k * 16
                for dc in range(16):
                    c = c0 + dc
                    for dd in range(d_model // LANES):
                        v = g[s, c, pl.ds(dd * LANES, LANES)]
                        plsc.store_scatter(tp, [base_idx[dd] + (off + c)], v)
                return carry

            lax.fori_loop(0, 8, body, 0)

        def fire_out(u, s, osem):
            h = u % hist
            j = jbase + u // hist
            for i in range(di):
                pltpu.async_copy(
                    tp.at[pl.ds(s * d_model * 128 + i * 1024, 1024)],
                    out_hbm.at[pl.ds(((h * di + i) * jt + j) * 1024, 1024)],
                    osem)

        def wait_out(osem):
            pltpu.make_async_copy(
                out_hbm.at[pl.ds(0, 1024)], tp.at[pl.ds(0, 1024)],
                osem).wait()

        gsems = (gs0, gs1, gs2, gs3)
        osems = (os0, os1)

        for u0 in range(3):
            build_idxcol(u0, u0)
            fire_gather(u0, gsems[u0])

        def quad(k, carry):
            u0 = 4 * k
            for s in range(4):
                u = u0 + s
                pf = (s + 3) % 4
                s2 = s % 2

                @pl.when(u + 3 < n_units)
                def _():
                    build_idxcol(u + 3, pf)
                    fire_gather(pf, gsems[pf])

                wait_gather(s, gsems[s])

                @pl.when(u >= 2)
                def _():
                    for _i in range(di):
                        wait_out(osems[s2])

                transpose(s, s2)
                fire_out(u, s2, osems[s2])
            return carry

        lax.fori_loop(0, n_quads, quad, 0)
        for s2 in range(2):
            for _i in range(di):
                wait_out(osems[s2])

    return gather_kernel


def kernel(x, table):
    b, h = x.shape
    d = table.shape[1]
    xp = jnp.pad(x.astype(jnp.int32), ((0, 0), (0, 128 - h))).reshape(-1)
    out1 = _make_sc_gather(b, h, d)(xp, table)
    out5 = out1.reshape(h, d // 8, b // 128, 8, 128)
    return out5.transpose(2, 4, 0, 1, 3).reshape(b, h, d)


# speedup vs baseline: 1.2995x; 1.2937x over previous
import functools

import jax
import jax.numpy as jnp
from jax import lax
from jax.experimental import pallas as pl
from jax.experimental.pallas import tpu as pltpu
from jax.experimental.pallas import tpu_sc as plsc

NC, NS = 2, 16
NW = NC * NS
NBUF = 8


@functools.lru_cache(maxsize=None)
def _make_sc_gather(b_total: int, hist: int, d_model: int):
    rows_w = b_total // NW
    n_groups = rows_w // NBUF
    n_pairs = n_groups // 2
    assert b_total == NW * rows_w and n_groups == 2 * n_pairs
    mesh = plsc.VectorSubcoreMesh(core_axis_name="c", subcore_axis_name="s")

    @functools.partial(
        pl.kernel,
        out_type=jax.ShapeDtypeStruct((b_total, hist, d_model), jnp.float32),
        mesh=mesh,
        scratch_types=[
            pltpu.VMEM((rows_w * 128,), jnp.int32),
            pltpu.VMEM((2, NBUF, hist, d_model), jnp.float32),
            pltpu.SemaphoreType.DMA,
            pltpu.SemaphoreType.DMA,
            pltpu.SemaphoreType.DMA,
            pltpu.SemaphoreType.DMA,
        ],
        compiler_params=pltpu.CompilerParams(use_tc_tiling_on_sc=False),
    )
    def gather_kernel(x_hbm, table_hbm, out_hbm, idx_v, rows_v, gs0, gs1,
                      os0, os1):
        wid = lax.axis_index("s") * NC + lax.axis_index("c")
        base = wid * rows_w
        pltpu.sync_copy(x_hbm.at[pl.ds(base * 128, rows_w * 128)], idx_v)

        def fire_gathers(g, slot, gsem):
            for b in range(NBUF):
                pltpu.async_copy(
                    table_hbm.at[idx_v.at[pl.ds((g * NBUF + b) * 128, hist)]],
                    rows_v.at[slot, b],
                    gsem)

        def wait_group(slot, sem):
            pltpu.make_async_copy(
                out_hbm.at[pl.ds(0, NBUF)], rows_v.at[slot], sem).wait()

        def fire_out(g, slot, osem):
            pltpu.async_copy(
                rows_v.at[slot],
                out_hbm.at[pl.ds(base + g * NBUF, NBUF)],
                osem)

        fire_gathers(0, 0, gs0)

        def pair(k, carry):
            a = 2 * k

            @pl.when(k > 0)
            def _():
                wait_group(1, os1)

            fire_gathers(a + 1, 1, gs1)
            wait_group(0, gs0)
            fire_out(a, 0, os0)
            wait_group(0, os0)

            @pl.when(k < n_pairs - 1)
            def _():
                fire_gathers(a + 2, 0, gs0)

            wait_group(1, gs1)
            fire_out(a + 1, 1, os1)
            return carry

        lax.fori_loop(0, n_pairs, pair, 0)
        wait_group(1, os1)

    return gather_kernel


def kernel(x, table):
    b, h = x.shape
    xp = jnp.pad(x.astype(jnp.int32), ((0, 0), (0, 128 - h))).reshape(-1)
    return _make_sc_gather(b, h, table.shape[1])(xp, table)
